# Initial kernel scaffold; baseline (speedup 1.0000x reference)
import jax
import jax.numpy as jnp
from jax.experimental import pallas as pl


def kernel(step, f_atoms, func2atom, mapping, W_i_atom, func_save):
    raise NotImplementedError("write your pallas kernel here")


# speedup vs baseline: 1.9540x; 1.9540x over previous
import functools

import jax
import jax.numpy as jnp
from jax import lax
from jax.experimental import pallas as pl
from jax.experimental.pallas import tpu as pltpu
from jax.experimental.pallas import tpu_sc as plsc

_N_ATOMS = 100000
_FDIM = 128
_HIDDEN = 300
_N_GROUPS = 20000
_GSIZE = 8
_N_FUNC = 64

_NW = 32
_K = 128
_NCHUNK = 40
_EPW = _K * _NCHUNK
_E_PAD = _NW * _EPW
_MROWS = 160


def _sc_body(f_hbm, a_hbm, b_hbm, z_hbm, out_hbm,
             aidx_v, bidx_v, rows_v, acc_sh, sem0, sem1):
    cid = lax.axis_index("c")
    sid = lax.axis_index("s")
    wid = sid * 2 + cid

    pltpu.sync_copy(a_hbm.at[wid], aidx_v)
    pltpu.sync_copy(b_hbm.at[wid], bidx_v)

    pltpu.async_copy(f_hbm.at[aidx_v.at[0]], rows_v.at[0], sem0)

    @pl.when(sid == 0)
    def _():
        pltpu.sync_copy(z_hbm, acc_sh)

    plsc.subcore_barrier()

    sems = (sem0, sem1)

    @pl.loop(0, _NCHUNK, step=2)
    def _(j):
        for b in range(2):
            i = j + b
            nxt = i + 1

            @pl.when(nxt < _NCHUNK)
            def _():
                pltpu.async_copy(f_hbm.at[aidx_v.at[nxt]],
                                 rows_v.at[(b + 1) % 2], sems[(b + 1) % 2])

            pltpu.make_async_copy(f_hbm.at[aidx_v.at[i]],
                                  rows_v.at[b], sems[b]).wait()
            pltpu.sync_copy(rows_v.at[b], acc_sh.at[bidx_v.at[i]], add=True)

    plsc.subcore_barrier()

    @pl.when(sid == 0)
    def _():
        pltpu.sync_copy(acc_sh, out_hbm.at[cid])


@functools.cache
def _sc_edge_sum():
    return functools.partial(
        pl.kernel,
        out_type=jax.ShapeDtypeStruct((2, _N_FUNC + 1, _FDIM), jnp.float32),
        mesh=plsc.VectorSubcoreMesh(core_axis_name="c", subcore_axis_name="s"),
        scratch_types=[
            pltpu.VMEM((_NCHUNK, _K), jnp.int32),
            pltpu.VMEM((_NCHUNK, _K), jnp.int32),
            pltpu.VMEM((2, _K, _FDIM), jnp.float32),
            pltpu.VMEM_SHARED((_N_FUNC + 1, _FDIM), jnp.float32),
            pltpu.SemaphoreType.DMA,
            pltpu.SemaphoreType.DMA,
        ],
    )(_sc_body)


def _tc_body(s2_ref, w_ref, fs_ref, m_ref, out_ref, cnt_ref):
    s = s2_ref[0, :_N_FUNC, :] + s2_ref[1, :_N_FUNC, :]
    out_ref[...] = fs_ref[...] + jnp.dot(
        s, w_ref[...], preferred_element_type=jnp.float32)

    iota = lax.broadcasted_iota(jnp.int32, (_N_FUNC, 128), 0)

    def body(r, acc):
        blk = m_ref[pl.ds(r, 1), :]
        return acc + (jnp.broadcast_to(blk, (_N_FUNC, 128)) == iota
                      ).astype(jnp.int32)

    acc = lax.fori_loop(0, _MROWS, body,
                        jnp.zeros((_N_FUNC, 128), jnp.int32))
    cnt_ref[...] = jnp.sum(acc, axis=1, keepdims=True) + 1


def kernel(step, f_atoms, func2atom, mapping, W_i_atom, func_save):
    del step
    a = func2atom.reshape(-1).astype(jnp.int32)
    b = jnp.repeat(mapping.astype(jnp.int32), _GSIZE)
    b = jnp.where(a == 0, _N_FUNC, b)
    a = jnp.maximum(a - 1, 0)
    pad = _E_PAD - a.shape[0]
    a = jnp.concatenate([a, jnp.zeros((pad,), jnp.int32)])
    b = jnp.concatenate([b, jnp.full((pad,), _N_FUNC, jnp.int32)])
    a3 = a.reshape(_NW, _NCHUNK, _K)
    b3 = b.reshape(_NW, _NCHUNK, _K)
    zeros = jnp.zeros((_N_FUNC + 1, _FDIM), jnp.float32)

    s2 = _sc_edge_sum()(f_atoms, a3, b3, zeros)

    mp = jnp.concatenate([
        mapping.astype(jnp.int32),
        jnp.full((_MROWS * 128 - _N_GROUPS,), -1, jnp.int32),
    ]).reshape(_MROWS, 128)

    func_save_new, cnt = pl.pallas_call(
        _tc_body,
        out_shape=(
            jax.ShapeDtypeStruct((_N_FUNC, _HIDDEN), jnp.float32),
            jax.ShapeDtypeStruct((_N_FUNC, 1), jnp.int32),
        ),
    )(s2, W_i_atom, func_save, mp)

    return func_save_new, cnt.reshape(_N_FUNC)
